# Initial kernel scaffold; baseline (speedup 1.0000x reference)
import jax
import jax.numpy as jnp
from jax.experimental import pallas as pl


def kernel(x, weight, self_loop_w, edge_index, edge_type):
    raise NotImplementedError("write your pallas kernel here")


# speedup vs baseline: 9.9048x; 9.9048x over previous
import functools

import jax
import jax.numpy as jnp
from jax import lax
from jax.experimental import pallas as pl
from jax.experimental.pallas import tpu as pltpu
from jax.experimental.pallas import tpu_sc as plsc

N_PAD = 10240
D = 128
NREL = 8
NC, NS, L = 2, 16, 16
NW = NC * NS
EDGES_PER_W = 5120
NCHUNK = 40
CHUNK = 128
ROWS_PER_TILE = N_PAD // NS


def _matmul_body(x_ref, w_ref, y_ref):
    y_ref[0] = jnp.dot(x_ref[...], w_ref[0], preferred_element_type=jnp.float32)


def _matmul(x_pad, w_all):
    nrb = N_PAD // 1024
    return pl.pallas_call(
        _matmul_body,
        grid=(nrb, w_all.shape[0]),
        in_specs=[
            pl.BlockSpec((1024, D), lambda i, r: (i, 0)),
            pl.BlockSpec((1, D, D), lambda i, r: (r, 0, 0)),
        ],
        out_specs=pl.BlockSpec((1, 1024, D), lambda i, r: (r, i, 0)),
        out_shape=jax.ShapeDtypeStruct((w_all.shape[0], N_PAD, D), jnp.float32),
    )(x_pad, w_all)


def _sc_body(table_hbm, src_hbm, typ_hbm, dst_hbm, out_hbm,
             gidx_v, typ_v, dst_v, rows_v, acc_sh, sem):
    cid = lax.axis_index("c")
    sid = lax.axis_index("s")
    wid = cid * NS + sid

    init_base = (NREL + cid) * N_PAD + sid * ROWS_PER_TILE
    pltpu.sync_copy(table_hbm.at[pl.ds(init_base, ROWS_PER_TILE)],
                    acc_sh.at[pl.ds(sid * ROWS_PER_TILE, ROWS_PER_TILE)])

    pltpu.sync_copy(src_hbm.at[wid], gidx_v)
    pltpu.sync_copy(typ_hbm.at[wid], typ_v)
    pltpu.sync_copy(dst_hbm.at[wid], dst_v)

    def _fuse_row(c, _):
        def _fuse16(j, _):
            sl = pl.ds(j * L, L)
            gidx_v[c, sl] = typ_v[c, sl] * N_PAD + gidx_v[c, sl]
            return 0
        return lax.fori_loop(0, CHUNK // L, _fuse16, 0)
    lax.fori_loop(0, NCHUNK, _fuse_row, 0)

    plsc.subcore_barrier()

    def _chunk(c, _):
        pltpu.async_copy(table_hbm.at[gidx_v.at[c]], rows_v, sem).wait()
        pltpu.sync_copy(rows_v, acc_sh.at[dst_v.at[c]], add=True)
        return 0
    lax.fori_loop(0, NCHUNK, _chunk, 0)

    plsc.subcore_barrier()
    pltpu.sync_copy(acc_sh.at[pl.ds(sid * ROWS_PER_TILE, ROWS_PER_TILE)],
                    out_hbm.at[cid, pl.ds(sid * ROWS_PER_TILE, ROWS_PER_TILE)])


_sc_scatter = functools.partial(
    pl.kernel,
    out_type=jax.ShapeDtypeStruct((NC, N_PAD, D), jnp.float32),
    mesh=plsc.VectorSubcoreMesh(core_axis_name="c", subcore_axis_name="s",
                                num_cores=NC, num_subcores=NS),
    scratch_types=[
        pltpu.VMEM((NCHUNK, CHUNK), jnp.int32),
        pltpu.VMEM((NCHUNK, CHUNK), jnp.int32),
        pltpu.VMEM((NCHUNK, CHUNK), jnp.int32),
        pltpu.VMEM((CHUNK, D), jnp.float32),
        pltpu.VMEM_SHARED((N_PAD, D), jnp.float32),
        pltpu.SemaphoreType.DMA,
    ],
)(_sc_body)


def _combine_body(p_ref, o_ref):
    o_ref[...] = jnp.maximum(p_ref[0] + p_ref[1], 0.0)


def _combine(partials):
    nrb = N_PAD // 1024
    return pl.pallas_call(
        _combine_body,
        grid=(nrb,),
        in_specs=[pl.BlockSpec((NC, 1024, D), lambda i: (0, i, 0))],
        out_specs=pl.BlockSpec((1024, D), lambda i: (i, 0)),
        out_shape=jax.ShapeDtypeStruct((N_PAD, D), jnp.float32),
    )(partials)


def kernel(x, weight, self_loop_w, edge_index, edge_type):
    n = x.shape[0]
    ne = edge_type.shape[0]
    x_pad = jnp.pad(x, ((0, N_PAD - n), (0, 0)))
    w_all = jnp.concatenate(
        [weight, self_loop_w.T[None], jnp.zeros((1, D, D), x.dtype)], axis=0)
    table = _matmul(x_pad, w_all).reshape(w_all.shape[0] * N_PAD, D)

    pad = NW * EDGES_PER_W - ne
    src_p = jnp.pad(edge_index[0], (0, pad)).reshape(NW, NCHUNK, CHUNK)
    typ_p = jnp.pad(edge_type, (0, pad)).reshape(NW, NCHUNK, CHUNK)
    dst_p = jnp.pad(edge_index[1], (0, pad),
                    constant_values=n).reshape(NW, NCHUNK, CHUNK)

    partials = _sc_scatter(table, src_p, typ_p, dst_p)
    return _combine(partials)[:n]


# speedup vs baseline: 10.0966x; 1.0194x over previous
import functools

import jax
import jax.numpy as jnp
from jax import lax
from jax.experimental import pallas as pl
from jax.experimental.pallas import tpu as pltpu
from jax.experimental.pallas import tpu_sc as plsc

N_PAD = 10240
D = 128
NREL = 8
NC, NS, L = 2, 16, 16
NW = NC * NS
EDGES_PER_W = 5120
NCHUNK = 40
CHUNK = 128
ROWS_PER_TILE = N_PAD // NS


def _matmul_body(x_ref, w_ref, y_ref):
    y_ref[0] = jnp.dot(x_ref[...], w_ref[0], preferred_element_type=jnp.float32)


def _matmul(x_pad, w_all):
    nrb = N_PAD // 1024
    return pl.pallas_call(
        _matmul_body,
        grid=(nrb, w_all.shape[0]),
        in_specs=[
            pl.BlockSpec((1024, D), lambda i, r: (i, 0)),
            pl.BlockSpec((1, D, D), lambda i, r: (r, 0, 0)),
        ],
        out_specs=pl.BlockSpec((1, 1024, D), lambda i, r: (r, i, 0)),
        out_shape=jax.ShapeDtypeStruct((w_all.shape[0], N_PAD, D), jnp.float32),
    )(x_pad, w_all)


NBUF = 2


def _sc_body(table_hbm, src_hbm, typ_hbm, dst_hbm, out_hbm,
             gidx_v, typ_v, dst_v, rows0, rows1, rows2, rows3, acc_sh,
             sem0, sem1, sem2, sem3):
    rows_b = (rows0, rows1, rows2, rows3)[:NBUF]
    sem_b = (sem0, sem1, sem2, sem3)[:NBUF]
    cid = lax.axis_index("c")
    sid = lax.axis_index("s")
    wid = cid * NS + sid

    init_base = (NREL + cid) * N_PAD + sid * ROWS_PER_TILE
    pltpu.sync_copy(table_hbm.at[pl.ds(init_base, ROWS_PER_TILE)],
                    acc_sh.at[pl.ds(sid * ROWS_PER_TILE, ROWS_PER_TILE)])

    pltpu.sync_copy(src_hbm.at[wid], gidx_v)
    pltpu.sync_copy(typ_hbm.at[wid], typ_v)
    pltpu.sync_copy(dst_hbm.at[wid], dst_v)

    def _fuse_row(c, _):
        def _fuse16(j, _):
            sl = pl.ds(j * L, L)
            gidx_v[c, sl] = typ_v[c, sl] * N_PAD + gidx_v[c, sl]
            return 0
        return lax.fori_loop(0, CHUNK // L, _fuse16, 0)
    lax.fori_loop(0, NCHUNK, _fuse_row, 0)

    plsc.subcore_barrier()

    def _grp(g, _):
        descs = []
        for b in range(NBUF):
            c = NBUF * g + b
            descs.append(pltpu.async_copy(
                table_hbm.at[gidx_v.at[c]], rows_b[b], sem_b[b]))
        for b in range(NBUF):
            c = NBUF * g + b
            descs[b].wait()
            pltpu.sync_copy(rows_b[b], acc_sh.at[dst_v.at[c]], add=True)
        return 0
    lax.fori_loop(0, NCHUNK // NBUF, _grp, 0)

    plsc.subcore_barrier()
    pltpu.sync_copy(acc_sh.at[pl.ds(sid * ROWS_PER_TILE, ROWS_PER_TILE)],
                    out_hbm.at[cid, pl.ds(sid * ROWS_PER_TILE, ROWS_PER_TILE)])


_sc_scatter = functools.partial(
    pl.kernel,
    out_type=jax.ShapeDtypeStruct((NC, N_PAD, D), jnp.float32),
    mesh=plsc.VectorSubcoreMesh(core_axis_name="c", subcore_axis_name="s",
                                num_cores=NC, num_subcores=NS),
    scratch_types=[
        pltpu.VMEM((NCHUNK, CHUNK), jnp.int32),
        pltpu.VMEM((NCHUNK, CHUNK), jnp.int32),
        pltpu.VMEM((NCHUNK, CHUNK), jnp.int32),
        pltpu.VMEM((CHUNK, D), jnp.float32),
        pltpu.VMEM((CHUNK, D), jnp.float32),
        pltpu.VMEM((CHUNK, D), jnp.float32),
        pltpu.VMEM((CHUNK, D), jnp.float32),
        pltpu.VMEM_SHARED((N_PAD, D), jnp.float32),
        pltpu.SemaphoreType.DMA,
        pltpu.SemaphoreType.DMA,
        pltpu.SemaphoreType.DMA,
        pltpu.SemaphoreType.DMA,
    ],
)(_sc_body)


def _combine_body(p_ref, o_ref):
    o_ref[...] = jnp.maximum(p_ref[0] + p_ref[1], 0.0)


def _combine(partials):
    nrb = N_PAD // 1024
    return pl.pallas_call(
        _combine_body,
        grid=(nrb,),
        in_specs=[pl.BlockSpec((NC, 1024, D), lambda i: (0, i, 0))],
        out_specs=pl.BlockSpec((1024, D), lambda i: (i, 0)),
        out_shape=jax.ShapeDtypeStruct((N_PAD, D), jnp.float32),
    )(partials)


def kernel(x, weight, self_loop_w, edge_index, edge_type):
    n = x.shape[0]
    ne = edge_type.shape[0]
    x_pad = jnp.pad(x, ((0, N_PAD - n), (0, 0)))
    w_all = jnp.concatenate(
        [weight, self_loop_w.T[None], jnp.zeros((1, D, D), x.dtype)], axis=0)
    table = _matmul(x_pad, w_all).reshape(w_all.shape[0] * N_PAD, D)

    pad = NW * EDGES_PER_W - ne
    src_p = jnp.pad(edge_index[0], (0, pad)).reshape(NW, NCHUNK, CHUNK)
    typ_p = jnp.pad(edge_type, (0, pad)).reshape(NW, NCHUNK, CHUNK)
    dst_p = jnp.pad(edge_index[1], (0, pad),
                    constant_values=n).reshape(NW, NCHUNK, CHUNK)

    partials = _sc_scatter(table, src_p, typ_p, dst_p)
    return _combine(partials)[:n]


# speedup vs baseline: 12.9726x; 1.2848x over previous
import functools

import jax
import jax.numpy as jnp
from jax import lax
from jax.experimental import pallas as pl
from jax.experimental.pallas import tpu as pltpu
from jax.experimental.pallas import tpu_sc as plsc

N_PAD = 10240
D = 128
NREL = 8
NC, NS, L = 2, 16, 16
NW = NC * NS
EDGES_PER_W = 5120
NCHUNK = 40
CHUNK = 128
ROWS_PER_TILE = N_PAD // NS


def _matmul_body(x_ref, w_ref, y_ref):
    y_ref[0] = jnp.dot(x_ref[...], w_ref[0], preferred_element_type=jnp.float32)


def _matmul(x_pad, w_all):
    nrb = N_PAD // 1024
    return pl.pallas_call(
        _matmul_body,
        grid=(nrb, w_all.shape[0]),
        in_specs=[
            pl.BlockSpec((1024, D), lambda i, r: (i, 0)),
            pl.BlockSpec((1, D, D), lambda i, r: (r, 0, 0)),
        ],
        out_specs=pl.BlockSpec((1, 1024, D), lambda i, r: (r, i, 0)),
        out_shape=jax.ShapeDtypeStruct((w_all.shape[0], N_PAD, D), jnp.float32),
    )(x_pad, w_all)


NBUF = 2
C0 = 56
C1 = 24
TOT_CHUNKS = NS * (C0 + C1)
ALLOC_CHUNKS = NS * C0 + (NS - 1) * C1 + C0


def _sc_body(table_hbm, gidx_hbm, dst_hbm, out_hbm,
             gidx_v, dst_v, rows0, rows1, acc_sh, sem0, sem1):
    rows_b = (rows0, rows1)
    sem_b = (sem0, sem1)
    cid = lax.axis_index("c")
    sid = lax.axis_index("s")

    init_base = (NREL + cid) * N_PAD + sid * ROWS_PER_TILE
    pltpu.sync_copy(table_hbm.at[pl.ds(init_base, ROWS_PER_TILE)],
                    acc_sh.at[pl.ds(sid * ROWS_PER_TILE, ROWS_PER_TILE)])
    plsc.subcore_barrier()

    nch = jnp.where(cid == 0, C0, C1)
    base = pl.multiple_of(jnp.where(cid == 0, sid * C0, NS * C0 + sid * C1), 8)

    pltpu.sync_copy(gidx_hbm.at[pl.ds(base, C0)], gidx_v)
    pltpu.sync_copy(dst_hbm.at[pl.ds(base, C0)], dst_v)

    def _grp(g, _):
        descs = []
        for b in range(NBUF):
            c = NBUF * g + b
            descs.append(pltpu.async_copy(
                table_hbm.at[gidx_v.at[c]], rows_b[b], sem_b[b]))
        for b in range(NBUF):
            c = NBUF * g + b
            descs[b].wait()
            pltpu.sync_copy(rows_b[b], acc_sh.at[dst_v.at[c]], add=True)
        return 0
    lax.fori_loop(0, nch // NBUF, _grp, 0)

    plsc.subcore_barrier()
    pltpu.sync_copy(acc_sh.at[pl.ds(sid * ROWS_PER_TILE, ROWS_PER_TILE)],
                    out_hbm.at[cid, pl.ds(sid * ROWS_PER_TILE, ROWS_PER_TILE)])


_sc_scatter = functools.partial(
    pl.kernel,
    out_type=jax.ShapeDtypeStruct((NC, N_PAD, D), jnp.float32),
    mesh=plsc.VectorSubcoreMesh(core_axis_name="c", subcore_axis_name="s",
                                num_cores=NC, num_subcores=NS),
    scratch_types=[
        pltpu.VMEM((C0, CHUNK), jnp.int32),
        pltpu.VMEM((C0, CHUNK), jnp.int32),
        pltpu.VMEM((CHUNK, D), jnp.float32),
        pltpu.VMEM((CHUNK, D), jnp.float32),
        pltpu.VMEM_SHARED((N_PAD, D), jnp.float32),
        pltpu.SemaphoreType.DMA,
        pltpu.SemaphoreType.DMA,
    ],
)(_sc_body)


def _combine_body(p_ref, o_ref):
    o_ref[...] = jnp.maximum(p_ref[0] + p_ref[1], 0.0)


def _combine(partials):
    nrb = N_PAD // 1024
    return pl.pallas_call(
        _combine_body,
        grid=(nrb,),
        in_specs=[pl.BlockSpec((NC, 1024, D), lambda i: (0, i, 0))],
        out_specs=pl.BlockSpec((1024, D), lambda i: (i, 0)),
        out_shape=jax.ShapeDtypeStruct((N_PAD, D), jnp.float32),
    )(partials)


def kernel(x, weight, self_loop_w, edge_index, edge_type):
    n = x.shape[0]
    ne = edge_type.shape[0]
    x_pad = jnp.pad(x, ((0, N_PAD - n), (0, 0)))
    w_all = jnp.concatenate(
        [weight, self_loop_w.T[None], jnp.zeros((1, D, D), x.dtype)], axis=0)
    table = _matmul(x_pad, w_all).reshape(w_all.shape[0] * N_PAD, D)

    pad = ALLOC_CHUNKS * CHUNK - ne
    gidx = edge_type * N_PAD + edge_index[0]
    gidx_p = jnp.pad(gidx, (0, pad)).reshape(ALLOC_CHUNKS, CHUNK)
    dst_p = jnp.pad(edge_index[1], (0, pad),
                    constant_values=n).reshape(ALLOC_CHUNKS, CHUNK)

    partials = _sc_scatter(table, gidx_p, dst_p)
    return _combine(partials)[:n]


# speedup vs baseline: 13.1099x; 1.0106x over previous
import functools

import jax
import jax.numpy as jnp
from jax import lax
from jax.experimental import pallas as pl
from jax.experimental.pallas import tpu as pltpu
from jax.experimental.pallas import tpu_sc as plsc

N_PAD = 10112
D = 128
NREL = 8
NC, NS, L = 2, 16, 16
CHUNK = 128
ROWS_PER_TILE = N_PAD // NS


def _matmul_body(x_ref, w_ref, y_ref):
    y_ref[0] = jnp.dot(x_ref[...], w_ref[0], preferred_element_type=jnp.float32)


MM_BLK = N_PAD // 8


def _matmul(x_pad, w_all):
    return pl.pallas_call(
        _matmul_body,
        grid=(8, w_all.shape[0]),
        in_specs=[
            pl.BlockSpec((MM_BLK, D), lambda i, r: (i, 0)),
            pl.BlockSpec((1, D, D), lambda i, r: (r, 0, 0)),
        ],
        out_specs=pl.BlockSpec((1, MM_BLK, D), lambda i, r: (r, i, 0)),
        out_shape=jax.ShapeDtypeStruct((w_all.shape[0], N_PAD, D), jnp.float32),
    )(x_pad, w_all)


NBUF = 2
C0 = 64
C1 = 16
TOT_CHUNKS = NS * (C0 + C1)
ALLOC_CHUNKS = NS * C0 + (NS - 1) * C1 + C0


def _sc_body(table_hbm, gidx_hbm, dst_hbm, out_hbm,
             gidx_v, dst_v, rows0, rows1, acc_sh, sem0, sem1):
    rows_b = (rows0, rows1)
    sem_b = (sem0, sem1)
    cid = lax.axis_index("c")
    sid = lax.axis_index("s")

    init_base = (NREL + cid) * N_PAD + sid * ROWS_PER_TILE
    pltpu.sync_copy(table_hbm.at[pl.ds(init_base, ROWS_PER_TILE)],
                    acc_sh.at[pl.ds(sid * ROWS_PER_TILE, ROWS_PER_TILE)])
    plsc.subcore_barrier()

    nch = jnp.where(cid == 0, C0, C1)
    base = pl.multiple_of(jnp.where(cid == 0, sid * C0, NS * C0 + sid * C1), 8)

    pltpu.sync_copy(gidx_hbm.at[pl.ds(base, C0)], gidx_v)
    pltpu.sync_copy(dst_hbm.at[pl.ds(base, C0)], dst_v)

    def _grp(g, _):
        descs = []
        for b in range(NBUF):
            c = NBUF * g + b
            descs.append(pltpu.async_copy(
                table_hbm.at[gidx_v.at[c]], rows_b[b], sem_b[b]))
        for b in range(NBUF):
            c = NBUF * g + b
            descs[b].wait()
            pltpu.sync_copy(rows_b[b], acc_sh.at[dst_v.at[c]], add=True)
        return 0
    lax.fori_loop(0, nch // NBUF, _grp, 0)

    plsc.subcore_barrier()
    pltpu.sync_copy(acc_sh.at[pl.ds(sid * ROWS_PER_TILE, ROWS_PER_TILE)],
                    out_hbm.at[cid, pl.ds(sid * ROWS_PER_TILE, ROWS_PER_TILE)])


_sc_scatter = functools.partial(
    pl.kernel,
    out_type=jax.ShapeDtypeStruct((NC, N_PAD, D), jnp.float32),
    mesh=plsc.VectorSubcoreMesh(core_axis_name="c", subcore_axis_name="s",
                                num_cores=NC, num_subcores=NS),
    scratch_types=[
        pltpu.VMEM((C0, CHUNK), jnp.int32),
        pltpu.VMEM((C0, CHUNK), jnp.int32),
        pltpu.VMEM((CHUNK, D), jnp.float32),
        pltpu.VMEM((CHUNK, D), jnp.float32),
        pltpu.VMEM_SHARED((N_PAD, D), jnp.float32),
        pltpu.SemaphoreType.DMA,
        pltpu.SemaphoreType.DMA,
    ],
)(_sc_body)


def _combine_body(p_ref, o_ref):
    o_ref[...] = jnp.maximum(p_ref[0] + p_ref[1], 0.0)


def _combine(partials, n):
    blk = n // 5
    return pl.pallas_call(
        _combine_body,
        grid=(5,),
        in_specs=[pl.BlockSpec((NC, blk, D), lambda i: (0, i, 0))],
        out_specs=pl.BlockSpec((blk, D), lambda i: (i, 0)),
        out_shape=jax.ShapeDtypeStruct((n, D), jnp.float32),
    )(partials)


def kernel(x, weight, self_loop_w, edge_index, edge_type):
    n = x.shape[0]
    ne = edge_type.shape[0]
    x_pad = jnp.pad(x, ((0, N_PAD - n), (0, 0)))
    w_all = jnp.concatenate(
        [weight, self_loop_w.T[None], jnp.zeros((1, D, D), x.dtype)], axis=0)
    table = _matmul(x_pad, w_all).reshape(w_all.shape[0] * N_PAD, D)

    pad = ALLOC_CHUNKS * CHUNK - ne
    gidx = edge_type * N_PAD + edge_index[0]
    gidx_p = jnp.pad(gidx, (0, pad)).reshape(ALLOC_CHUNKS, CHUNK)
    dst_p = jnp.pad(edge_index[1], (0, pad),
                    constant_values=n).reshape(ALLOC_CHUNKS, CHUNK)

    partials = _sc_scatter(table, gidx_p, dst_p)
    return _combine(partials, n)
